# baseline (speedup 1.0000x reference)
import functools

import jax
import jax.numpy as jnp
from jax import lax
from jax.experimental import pallas as pl
from jax.experimental.pallas import tpu as pltpu
from jax.experimental.pallas import tpu_sc as plsc

N = 100000
NP = 100352
NPR = NP // 128
RW = NP // 32
RPT = RW // 16
E = 6400000
G = 1000
GP = 1024
NC, NS = 2, 16
NW = NC * NS
EPT = E // NW
TSL = NP // NW
CH_A = 4000
CH_C = 800
LANES = 16

_mesh = plsc.VectorSubcoreMesh(core_axis_name="c", subcore_axis_name="s")
_sc_params = pltpu.CompilerParams(
    needs_layout_passes=False, use_tc_tiling_on_sc=False)


def _zero_ref1d(ref, n):
    z = jnp.zeros((LANES,), jnp.float32)

    def body(i, _):
        ref[pl.ds(i * LANES, LANES)] = z
        return 0

    lax.fori_loop(0, n // LANES, body, 0)


def _reduce_to_hbm(local2d, sh_acc, rbuf, idx_tbl, out_hbm, cid, sid):
    plsc.subcore_barrier()
    for rr in range(NS):
        r = sid + rr
        r = jnp.where(r >= NS, r - NS, r)
        pltpu.sync_copy(
            local2d.at[pl.ds(r * RPT, RPT)],
            sh_acc.at[idx_tbl.at[r]],
            add=True,
        )
    plsc.subcore_barrier()
    pltpu.sync_copy(sh_acc.at[pl.ds(sid * RPT, RPT)], rbuf)
    pltpu.sync_copy(rbuf, out_hbm.at[cid, pl.ds(sid * RPT, RPT)])


@functools.partial(
    pl.kernel,
    out_type=jax.ShapeDtypeStruct((NC, RW, 32), jnp.float32),
    mesh=_mesh,
    compiler_params=_sc_params,
    scratch_types=[
        pltpu.VMEM((RW, 32), jnp.float32),
        pltpu.VMEM((CH_A,), jnp.int32),
        pltpu.VMEM((CH_A,), jnp.int32),
        pltpu.VMEM((RPT, 32), jnp.float32),
        pltpu.VMEM((NS, RPT), jnp.int32),
        pltpu.VMEM_SHARED((RW, 32), jnp.float32),
        pltpu.SemaphoreType.DMA,
        pltpu.SemaphoreType.DMA,
    ],
)
def _deg_kernel(dst_hbm, zeros_hbm, iota_hbm, out_hbm,
                hist, db0, db1, rbuf, idx_tbl, sh_acc, s0, s1):
    cid = lax.axis_index("c")
    sid = lax.axis_index("s")
    base = (cid * NS + sid) * EPT
    ones = jnp.full((LANES,), 1.0, jnp.float32)
    five = jnp.full((LANES,), 5, jnp.int32)
    m31 = jnp.full((LANES,), 31, jnp.int32)
    nch = EPT // CH_A

    pltpu.sync_copy(zeros_hbm, hist)
    pltpu.sync_copy(iota_hbm, idx_tbl)
    pltpu.sync_copy(zeros_hbm.at[pl.ds(sid * RPT, RPT)],
                    sh_acc.at[pl.ds(sid * RPT, RPT)])

    bufs = (db0, db1)
    sems = (s0, s1)
    for b in range(2):
        pltpu.make_async_copy(
            dst_hbm.at[pl.ds(base + b * CH_A, CH_A)], bufs[b], sems[b]
        ).start()

    def outer(k2, _):
        for b in range(2):
            k = k2 * 2 + b
            pltpu.make_async_copy(
                dst_hbm.at[pl.ds(base + k * CH_A, CH_A)], bufs[b], sems[b]
            ).wait()

            def inner(i, _):
                idx = bufs[b][pl.ds(i * LANES, LANES)]
                plsc.addupdate_scatter(
                    hist,
                    [lax.shift_right_logical(idx, five), idx & m31],
                    ones,
                )
                return 0

            lax.fori_loop(0, CH_A // LANES, inner, 0, unroll=10)

            @pl.when(k + 2 < nch)
            def _():
                pltpu.make_async_copy(
                    dst_hbm.at[pl.ds(base + (k + 2) * CH_A, CH_A)],
                    bufs[b],
                    sems[b],
                ).start()

        return 0

    lax.fori_loop(0, nch // 2, outer, 0)
    _reduce_to_hbm(hist, sh_acc, rbuf, idx_tbl, out_hbm, cid, sid)


@functools.partial(
    pl.kernel,
    out_type=jax.ShapeDtypeStruct((NC, RW, 32), jnp.float32),
    mesh=_mesh,
    compiler_params=_sc_params,
    scratch_types=[
        pltpu.VMEM((RW, 32), jnp.float32),
        pltpu.VMEM((CH_C,), jnp.int32),
        pltpu.VMEM((CH_C,), jnp.int32),
        pltpu.VMEM((CH_C,), jnp.int32),
        pltpu.VMEM((CH_C,), jnp.int32),
        pltpu.VMEM((CH_C,), jnp.float32),
        pltpu.VMEM((CH_C,), jnp.float32),
        pltpu.VMEM((RPT, 32), jnp.float32),
        pltpu.VMEM((NS, RPT), jnp.int32),
        pltpu.VMEM_SHARED((NP,), jnp.float32),
        pltpu.VMEM_SHARED((RW, 32), jnp.float32),
        pltpu.SemaphoreType.DMA,
        pltpu.SemaphoreType.DMA,
        pltpu.SemaphoreType.DMA,
        pltpu.SemaphoreType.DMA,
        pltpu.SemaphoreType.DMA,
        pltpu.SemaphoreType.DMA,
    ],
)
def _msg_kernel(src_hbm, dst_hbm, m_hbm, zeros_hbm, iota_hbm, out_hbm,
                accA, sb0, sb1, db0, db1, vb0, vb1, rbuf, idx_tbl,
                sh_m, sh_acc, ss0, ss1, sd0, sd1, sv0, sv1):
    cid = lax.axis_index("c")
    sid = lax.axis_index("s")
    base = (cid * NS + sid) * EPT
    five = jnp.full((LANES,), 5, jnp.int32)
    m31 = jnp.full((LANES,), 31, jnp.int32)
    nch = EPT // CH_C

    pltpu.sync_copy(m_hbm.at[pl.ds(sid * (NP // NS), NP // NS)],
                    sh_m.at[pl.ds(sid * (NP // NS), NP // NS)])
    pltpu.sync_copy(zeros_hbm, accA)
    pltpu.sync_copy(iota_hbm, idx_tbl)
    pltpu.sync_copy(zeros_hbm.at[pl.ds(sid * RPT, RPT)],
                    sh_acc.at[pl.ds(sid * RPT, RPT)])
    plsc.subcore_barrier()

    sbufs = (sb0, sb1)
    dbufs = (db0, db1)
    vbufs = (vb0, vb1)
    ssems = (ss0, ss1)
    dsems = (sd0, sd1)
    vsems = (sv0, sv1)

    def src_cp(k, b):
        return pltpu.make_async_copy(
            src_hbm.at[pl.ds(base + k * CH_C, CH_C)], sbufs[b], ssems[b])

    def dst_cp(k, b):
        return pltpu.make_async_copy(
            dst_hbm.at[pl.ds(base + k * CH_C, CH_C)], dbufs[b], dsems[b])

    def gather_cp(b):
        return pltpu.make_async_copy(sh_m.at[sbufs[b]], vbufs[b], vsems[b])

    src_cp(0, 0).start()
    dst_cp(0, 0).start()
    dst_cp(1, 1).start()
    src_cp(0, 0).wait()
    gather_cp(0).start()
    src_cp(1, 1).start()

    def outer(k2, _):
        for b in range(2):
            k = k2 * 2 + b
            ob = 1 - b
            gather_cp(b).wait()

            @pl.when(k + 1 < nch)
            def _():
                src_cp(k + 1, ob).wait()
                gather_cp(ob).start()

            @pl.when(k + 2 < nch)
            def _():
                src_cp(k + 2, b).start()

            dst_cp(k, b).wait()

            def inner(i, _):
                sl = pl.ds(i * LANES, LANES)
                d = dbufs[b][sl]
                v = vbufs[b][sl]
                plsc.addupdate_scatter(
                    accA,
                    [lax.shift_right_logical(d, five), d & m31],
                    v,
                )
                return 0

            lax.fori_loop(0, CH_C // LANES, inner, 0, unroll=10)

            @pl.when(k + 2 < nch)
            def _():
                dst_cp(k + 2, b).start()

        return 0

    lax.fori_loop(0, nch // 2, outer, 0)
    _reduce_to_hbm(accA, sh_acc, rbuf, idx_tbl, out_hbm, cid, sid)


@functools.partial(
    pl.kernel,
    out_type=jax.ShapeDtypeStruct((NW, 2 * GP), jnp.float32),
    mesh=_mesh,
    compiler_params=_sc_params,
    scratch_types=[
        pltpu.VMEM((2 * GP,), jnp.float32),
        pltpu.VMEM((TSL,), jnp.float32),
        pltpu.VMEM((TSL,), jnp.float32),
        pltpu.VMEM((TSL,), jnp.float32),
        pltpu.VMEM((TSL,), jnp.float32),
        pltpu.VMEM((TSL,), jnp.int32),
    ],
)
def _graph_kernel(dinv_hbm, m_hbm, a2_hbm, batch_hbm, out_hbm,
                  hist, bd, bm, ba0, ba1, bb):
    cid = lax.axis_index("c")
    sid = lax.axis_index("s")
    wid = cid * NS + sid
    off = wid * TSL
    pltpu.sync_copy(dinv_hbm.at[pl.ds(off, TSL)], bd)
    pltpu.sync_copy(m_hbm.at[pl.ds(off, TSL)], bm)
    pltpu.sync_copy(a2_hbm.at[0, pl.ds(off, TSL)], ba0)
    pltpu.sync_copy(a2_hbm.at[1, pl.ds(off, TSL)], ba1)
    pltpu.sync_copy(batch_hbm.at[pl.ds(off, TSL)], bb)
    _zero_ref1d(hist, 2 * GP)
    ones = jnp.full((LANES,), 1.0, jnp.float32)
    gofs = jnp.full((LANES,), GP, jnp.int32)

    def body(i, _):
        sl = pl.ds(i * LANES, LANES)
        val = bd[sl] * (ba0[sl] + ba1[sl] + bm[sl])
        b = bb[sl]
        plsc.addupdate_scatter(hist, [b], val)
        plsc.addupdate_scatter(hist, [b + gofs], ones)
        return 0

    lax.fori_loop(0, TSL // LANES, body, 0, unroll=7)
    pltpu.sync_copy(hist, out_hbm.at[wid])


def _node_body(xc_ref, dg_ref, wlf_ref, blf_ref, wg_ref, bg_ref,
               m_ref, dinv_ref):
    c = [wg_ref[1 + i, 0] for i in range(4)]
    bg = bg_ref[0]
    wg0 = wg_ref[0, 0]
    xs = [xc_ref[d] for d in range(5)]
    h = bg + wg0 * xs[0]
    for j in range(4):
        gj = sum(c[i] * blf_ref[4 * i + j] for i in range(4))
        acc = c[j] + gj
        t = acc * xs[1 + j]
        for d in range(5):
            adj = sum(c[i] * wlf_ref[d, 4 * i + j] for i in range(4))
            t = t + (adj * xs[d]) * xs[1 + j]
        h = h + t
    deg = dg_ref[0] + dg_ref[1] + 1.0
    dinv = lax.rsqrt(deg)
    m_ref[...] = h * dinv
    dinv_ref[...] = dinv


def _combine_body(num_ref, cnt_ref, out_ref):
    num = jnp.sum(num_ref[...], axis=0, keepdims=True)
    cnt = jnp.sum(cnt_ref[...], axis=0, keepdims=True)
    out_ref[...] = num / cnt


def kernel(x, edge_index, batch, W_lf, b_lf, W_gcn, b_gcn):
    src = edge_index[0]
    dst = edge_index[1]
    xp = jnp.pad(x, ((0, NP - N), (0, 0)))
    xcols = xp.T.reshape(5, NPR, 128)
    batchp = jnp.pad(batch, (0, NP - N), constant_values=G)
    zeros2d = jnp.zeros((RW, 32), jnp.float32)
    iota_tbl = jnp.arange(RW, dtype=jnp.int32).reshape(NS, RPT)

    deg2 = _deg_kernel(dst, zeros2d, iota_tbl)

    m2d, dinv2d = pl.pallas_call(
        _node_body,
        grid=(NPR // 8,),
        in_specs=[
            pl.BlockSpec((5, 8, 128), lambda i: (0, i, 0)),
            pl.BlockSpec((2, 8, 128), lambda i: (0, i, 0)),
            pl.BlockSpec(memory_space=pltpu.SMEM),
            pl.BlockSpec(memory_space=pltpu.SMEM),
            pl.BlockSpec(memory_space=pltpu.SMEM),
            pl.BlockSpec(memory_space=pltpu.SMEM),
        ],
        out_specs=[
            pl.BlockSpec((8, 128), lambda i: (i, 0)),
            pl.BlockSpec((8, 128), lambda i: (i, 0)),
        ],
        out_shape=[
            jax.ShapeDtypeStruct((NPR, 128), jnp.float32),
            jax.ShapeDtypeStruct((NPR, 128), jnp.float32),
        ],
    )(xcols, deg2.reshape(NC, NPR, 128), W_lf, b_lf, W_gcn, b_gcn)

    m_flat = m2d.reshape(NP)
    dinv_flat = dinv2d.reshape(NP)

    a2 = _msg_kernel(src, dst, m_flat, zeros2d, iota_tbl)
    part = _graph_kernel(dinv_flat, m_flat, a2.reshape(NC, NP), batchp)

    out = pl.pallas_call(
        _combine_body,
        out_shape=jax.ShapeDtypeStruct((1, GP), jnp.float32),
    )(part[:, :GP], part[:, GP:])
    return out[0, :G]


# speedup vs baseline: 1.1852x; 1.1852x over previous
import functools

import jax
import jax.numpy as jnp
from jax import lax
from jax.experimental import pallas as pl
from jax.experimental.pallas import tpu as pltpu
from jax.experimental.pallas import tpu_sc as plsc

N = 100000
NP = 100352
NPR = NP // 128
RW = NP // 32
RPT = RW // 16
E = 6400000
G = 1000
GP = 1024
NC, NS = 2, 16
NW = NC * NS
EPT = E // NW
TSL = NP // NW
CH_A = 10000
CH_C = 2000
LANES = 16

_mesh = plsc.VectorSubcoreMesh(core_axis_name="c", subcore_axis_name="s")
_sc_params = pltpu.CompilerParams(
    needs_layout_passes=False, use_tc_tiling_on_sc=False)


def _zero_ref1d(ref, n):
    z = jnp.zeros((LANES,), jnp.float32)

    def body(i, _):
        ref[pl.ds(i * LANES, LANES)] = z
        return 0

    lax.fori_loop(0, n // LANES, body, 0)


def _reduce_to_hbm(local2d, sh_acc, idx_tbl, out_hbm, cid, sid):
    plsc.subcore_barrier()
    for rr in range(NS):
        r = sid + rr
        r = jnp.where(r >= NS, r - NS, r)
        pltpu.sync_copy(
            local2d.at[pl.ds(r * RPT, RPT)],
            sh_acc.at[idx_tbl.at[r]],
            add=True,
        )
    plsc.subcore_barrier()
    pltpu.sync_copy(sh_acc.at[pl.ds(sid * RPT, RPT)],
                    out_hbm.at[cid, pl.ds(sid * RPT, RPT)])


@functools.partial(
    pl.kernel,
    out_type=jax.ShapeDtypeStruct((NC, RW, 32), jnp.float32),
    mesh=_mesh,
    compiler_params=_sc_params,
    scratch_types=[
        pltpu.VMEM((RW, 32), jnp.float32),
        pltpu.VMEM((CH_A,), jnp.int32),
        pltpu.VMEM((CH_A,), jnp.int32),
        pltpu.VMEM((NS, RPT), jnp.int32),
        pltpu.VMEM_SHARED((RW, 32), jnp.float32),
        pltpu.SemaphoreType.DMA,
        pltpu.SemaphoreType.DMA,
    ],
)
def _deg_kernel(dst_hbm, zeros_hbm, iota_hbm, out_hbm,
                hist, db0, db1, idx_tbl, sh_acc, s0, s1):
    cid = lax.axis_index("c")
    sid = lax.axis_index("s")
    base = (cid * NS + sid) * EPT
    ones = jnp.full((LANES,), 1.0, jnp.float32)
    five = jnp.full((LANES,), 5, jnp.int32)
    m31 = jnp.full((LANES,), 31, jnp.int32)
    nch = EPT // CH_A

    pltpu.sync_copy(zeros_hbm, hist)
    pltpu.sync_copy(iota_hbm, idx_tbl)
    pltpu.sync_copy(zeros_hbm.at[pl.ds(sid * RPT, RPT)],
                    sh_acc.at[pl.ds(sid * RPT, RPT)])

    bufs = (db0, db1)
    sems = (s0, s1)
    for b in range(2):
        pltpu.make_async_copy(
            dst_hbm.at[pl.ds(base + b * CH_A, CH_A)], bufs[b], sems[b]
        ).start()

    def outer(k2, _):
        for b in range(2):
            k = k2 * 2 + b
            pltpu.make_async_copy(
                dst_hbm.at[pl.ds(base + k * CH_A, CH_A)], bufs[b], sems[b]
            ).wait()

            def inner(i, _):
                idx = bufs[b][pl.ds(i * LANES, LANES)]
                plsc.addupdate_scatter(
                    hist,
                    [lax.shift_right_logical(idx, five), idx & m31],
                    ones,
                )
                return 0

            lax.fori_loop(0, CH_A // LANES, inner, 0, unroll=10)

            @pl.when(k + 2 < nch)
            def _():
                pltpu.make_async_copy(
                    dst_hbm.at[pl.ds(base + (k + 2) * CH_A, CH_A)],
                    bufs[b],
                    sems[b],
                ).start()

        return 0

    lax.fori_loop(0, nch // 2, outer, 0)
    _reduce_to_hbm(hist, sh_acc, idx_tbl, out_hbm, cid, sid)


@functools.partial(
    pl.kernel,
    out_type=jax.ShapeDtypeStruct((NC, RW, 32), jnp.float32),
    mesh=_mesh,
    compiler_params=_sc_params,
    scratch_types=[
        pltpu.VMEM((RW, 32), jnp.float32),
        pltpu.VMEM((CH_C,), jnp.int32),
        pltpu.VMEM((CH_C,), jnp.int32),
        pltpu.VMEM((CH_C,), jnp.int32),
        pltpu.VMEM((CH_C,), jnp.int32),
        pltpu.VMEM((CH_C,), jnp.float32),
        pltpu.VMEM((CH_C,), jnp.float32),
        pltpu.VMEM((NS, RPT), jnp.int32),
        pltpu.VMEM_SHARED((NP,), jnp.float32),
        pltpu.VMEM_SHARED((RW, 32), jnp.float32),
        pltpu.SemaphoreType.DMA,
        pltpu.SemaphoreType.DMA,
        pltpu.SemaphoreType.DMA,
        pltpu.SemaphoreType.DMA,
        pltpu.SemaphoreType.DMA,
        pltpu.SemaphoreType.DMA,
    ],
)
def _msg_kernel(src_hbm, dst_hbm, m_hbm, zeros_hbm, iota_hbm, out_hbm,
                accA, sb0, sb1, db0, db1, vb0, vb1, idx_tbl,
                sh_m, sh_acc, ss0, ss1, sd0, sd1, sv0, sv1):
    cid = lax.axis_index("c")
    sid = lax.axis_index("s")
    base = (cid * NS + sid) * EPT
    five = jnp.full((LANES,), 5, jnp.int32)
    m31 = jnp.full((LANES,), 31, jnp.int32)
    nch = EPT // CH_C

    pltpu.sync_copy(m_hbm.at[pl.ds(sid * (NP // NS), NP // NS)],
                    sh_m.at[pl.ds(sid * (NP // NS), NP // NS)])
    pltpu.sync_copy(zeros_hbm, accA)
    pltpu.sync_copy(iota_hbm, idx_tbl)
    pltpu.sync_copy(zeros_hbm.at[pl.ds(sid * RPT, RPT)],
                    sh_acc.at[pl.ds(sid * RPT, RPT)])
    plsc.subcore_barrier()

    sbufs = (sb0, sb1)
    dbufs = (db0, db1)
    vbufs = (vb0, vb1)
    ssems = (ss0, ss1)
    dsems = (sd0, sd1)
    vsems = (sv0, sv1)

    def src_cp(k, b):
        return pltpu.make_async_copy(
            src_hbm.at[pl.ds(base + k * CH_C, CH_C)], sbufs[b], ssems[b])

    def dst_cp(k, b):
        return pltpu.make_async_copy(
            dst_hbm.at[pl.ds(base + k * CH_C, CH_C)], dbufs[b], dsems[b])

    def gather_cp(b):
        return pltpu.make_async_copy(sh_m.at[sbufs[b]], vbufs[b], vsems[b])

    src_cp(0, 0).start()
    dst_cp(0, 0).start()
    dst_cp(1, 1).start()
    src_cp(0, 0).wait()
    gather_cp(0).start()
    src_cp(1, 1).start()

    def outer(k2, _):
        for b in range(2):
            k = k2 * 2 + b
            ob = 1 - b
            gather_cp(b).wait()

            @pl.when(k + 1 < nch)
            def _():
                src_cp(k + 1, ob).wait()
                gather_cp(ob).start()

            @pl.when(k + 2 < nch)
            def _():
                src_cp(k + 2, b).start()

            dst_cp(k, b).wait()

            def inner(i, _):
                sl = pl.ds(i * LANES, LANES)
                d = dbufs[b][sl]
                v = vbufs[b][sl]
                plsc.addupdate_scatter(
                    accA,
                    [lax.shift_right_logical(d, five), d & m31],
                    v,
                )
                return 0

            lax.fori_loop(0, CH_C // LANES, inner, 0, unroll=10)

            @pl.when(k + 2 < nch)
            def _():
                dst_cp(k + 2, b).start()

        return 0

    lax.fori_loop(0, nch // 2, outer, 0)
    _reduce_to_hbm(accA, sh_acc, idx_tbl, out_hbm, cid, sid)


@functools.partial(
    pl.kernel,
    out_type=jax.ShapeDtypeStruct((NW, 2 * GP), jnp.float32),
    mesh=_mesh,
    compiler_params=_sc_params,
    scratch_types=[
        pltpu.VMEM((2 * GP,), jnp.float32),
        pltpu.VMEM((TSL,), jnp.float32),
        pltpu.VMEM((TSL,), jnp.float32),
        pltpu.VMEM((TSL,), jnp.float32),
        pltpu.VMEM((TSL,), jnp.float32),
        pltpu.VMEM((TSL,), jnp.int32),
    ],
)
def _graph_kernel(dinv_hbm, m_hbm, a2_hbm, batch_hbm, out_hbm,
                  hist, bd, bm, ba0, ba1, bb):
    cid = lax.axis_index("c")
    sid = lax.axis_index("s")
    wid = cid * NS + sid
    off = wid * TSL
    pltpu.sync_copy(dinv_hbm.at[pl.ds(off, TSL)], bd)
    pltpu.sync_copy(m_hbm.at[pl.ds(off, TSL)], bm)
    pltpu.sync_copy(a2_hbm.at[0, pl.ds(off, TSL)], ba0)
    pltpu.sync_copy(a2_hbm.at[1, pl.ds(off, TSL)], ba1)
    pltpu.sync_copy(batch_hbm.at[pl.ds(off, TSL)], bb)
    _zero_ref1d(hist, 2 * GP)
    ones = jnp.full((LANES,), 1.0, jnp.float32)
    gofs = jnp.full((LANES,), GP, jnp.int32)

    def body(i, _):
        sl = pl.ds(i * LANES, LANES)
        val = bd[sl] * (ba0[sl] + ba1[sl] + bm[sl])
        b = bb[sl]
        plsc.addupdate_scatter(hist, [b], val)
        plsc.addupdate_scatter(hist, [b + gofs], ones)
        return 0

    lax.fori_loop(0, TSL // LANES, body, 0, unroll=7)
    pltpu.sync_copy(hist, out_hbm.at[wid])


def _node_body(xc_ref, dg_ref, wlf_ref, blf_ref, wg_ref, bg_ref,
               m_ref, dinv_ref):
    c = [wg_ref[1 + i, 0] for i in range(4)]
    bg = bg_ref[0]
    wg0 = wg_ref[0, 0]
    xs = [xc_ref[d] for d in range(5)]
    h = bg + wg0 * xs[0]
    for j in range(4):
        gj = sum(c[i] * blf_ref[4 * i + j] for i in range(4))
        acc = c[j] + gj
        t = acc * xs[1 + j]
        for d in range(5):
            adj = sum(c[i] * wlf_ref[d, 4 * i + j] for i in range(4))
            t = t + (adj * xs[d]) * xs[1 + j]
        h = h + t
    deg = dg_ref[0] + dg_ref[1] + 1.0
    dinv = lax.rsqrt(deg)
    m_ref[...] = h * dinv
    dinv_ref[...] = dinv


def _combine_body(num_ref, cnt_ref, out_ref):
    num = jnp.sum(num_ref[...], axis=0, keepdims=True)
    cnt = jnp.sum(cnt_ref[...], axis=0, keepdims=True)
    out_ref[...] = num / cnt


def kernel(x, edge_index, batch, W_lf, b_lf, W_gcn, b_gcn):
    src = edge_index[0]
    dst = edge_index[1]
    xp = jnp.pad(x, ((0, NP - N), (0, 0)))
    xcols = xp.T.reshape(5, NPR, 128)
    batchp = jnp.pad(batch, (0, NP - N), constant_values=G)
    zeros2d = jnp.zeros((RW, 32), jnp.float32)
    iota_tbl = jnp.arange(RW, dtype=jnp.int32).reshape(NS, RPT)

    deg2 = _deg_kernel(dst, zeros2d, iota_tbl)

    m2d, dinv2d = pl.pallas_call(
        _node_body,
        grid=(NPR // 8,),
        in_specs=[
            pl.BlockSpec((5, 8, 128), lambda i: (0, i, 0)),
            pl.BlockSpec((2, 8, 128), lambda i: (0, i, 0)),
            pl.BlockSpec(memory_space=pltpu.SMEM),
            pl.BlockSpec(memory_space=pltpu.SMEM),
            pl.BlockSpec(memory_space=pltpu.SMEM),
            pl.BlockSpec(memory_space=pltpu.SMEM),
        ],
        out_specs=[
            pl.BlockSpec((8, 128), lambda i: (i, 0)),
            pl.BlockSpec((8, 128), lambda i: (i, 0)),
        ],
        out_shape=[
            jax.ShapeDtypeStruct((NPR, 128), jnp.float32),
            jax.ShapeDtypeStruct((NPR, 128), jnp.float32),
        ],
    )(xcols, deg2.reshape(NC, NPR, 128), W_lf, b_lf, W_gcn, b_gcn)

    m_flat = m2d.reshape(NP)
    dinv_flat = dinv2d.reshape(NP)

    a2 = _msg_kernel(src, dst, m_flat, zeros2d, iota_tbl)
    part = _graph_kernel(dinv_flat, m_flat, a2.reshape(NC, NP), batchp)

    out = pl.pallas_call(
        _combine_body,
        out_shape=jax.ShapeDtypeStruct((1, GP), jnp.float32),
    )(part[:, :GP], part[:, GP:])
    return out[0, :G]
